# baseline (speedup 1.0000x reference)
import math

import jax
import jax.numpy as jnp
from jax import lax
from jax.experimental import pallas as pl

_B, _L, _H = 4096, 200, 64
_NT = _B * _L
_BL = 2048
_RPB = 4
_GRID = _NT // (2 * _BL * _RPB)
_OROWS = _NT * _H // (8 * 128)


def _fuse_body(e_ref, w_ref, b_ref, be_ref, g_ref, mt_ref, bg_ref):
    mt = lax.dot_general(w_ref[...], e_ref[...], (((0,), (1,)), ((), ())),
                         preferred_element_type=jnp.float32)
    sel = (lax.broadcasted_iota(jnp.int32, (64, 64), 1) == 7).astype(jnp.float32)
    mt = mt + sel * b_ref[...]
    mt_ref[...] = mt - jnp.mean(mt, axis=0, keepdims=True)
    bg_ref[...] = be_ref[...] / g_ref[...]


def _half(wd, sm, du, td, mt, bins, bg, k):
    bf = jnp.bfloat16
    hr = jnp.clip(sm // 60, 0, 23)
    theta = sm.astype(jnp.float32) * jnp.float32(2.0 * math.pi / 1440.0)
    sinv = jnp.sin(theta).astype(bf)
    cosv = jnp.cos(theta).astype(bf)
    ld = jnp.log1p(du)
    cnt = jnp.sum((bins < ld).astype(jnp.int32), axis=0, keepdims=True)
    db = jnp.clip(cnt - 1, 0, 9)
    wd_b = wd.astype(bf)
    hr_b = (hr + 8).astype(bf)
    db_b = (db + 32).astype(bf)
    td_b = (td + 42).astype(bf)
    k0 = k[0:16, :]
    k1 = k[16:32, :]
    k2 = k[32:48, :]
    k3 = k[48:64, :]
    s0 = ((k0 == wd_b) | (k0 == bf(7.0)) | (k0 == hr_b)).astype(bf)
    s1 = (k1 == hr_b).astype(bf)
    s2 = ((k2 == db_b) | (k2 == td_b)).astype(bf)
    s3 = jnp.where(k3 == td_b, bf(1.0),
                   jnp.where(k3 == bf(50.0), sinv,
                             jnp.where(k3 == bf(51.0), cosv, bf(0.0))))
    oh = jnp.concatenate([s0, s1, s2, s3], axis=0)
    ht = lax.dot_general(mt, oh, (((1,), (0,)), ((), ())),
                         preferred_element_type=jnp.float32)
    var = jnp.sum(ht * ht, axis=0, keepdims=True) * jnp.float32(1.0 / 64.0)
    inv = lax.rsqrt(var + 1e-5)
    return ht * inv + bg


def _main_body(wde_ref, wdo_ref, sme_ref, smo_ref, due_ref, duo_ref,
               tde_ref, tdo_ref, mt_ref, bins_ref, gd2_ref, bg_ref, o_ref):
    mt = mt_ref[...].astype(jnp.bfloat16)
    gd2 = gd2_ref[...]
    bg = bg_ref[...]
    bins = bins_ref[...]
    k = lax.broadcasted_iota(jnp.int16, (64, _BL), 0).astype(jnp.bfloat16)
    for r in range(_RPB):
        ye = _half(wde_ref[0, r:r + 1, :], sme_ref[0, r:r + 1, :],
                   due_ref[0, r:r + 1, :], tde_ref[0, r:r + 1, :],
                   mt, bins, bg, k)
        yo = _half(wdo_ref[0, r:r + 1, :], smo_ref[0, r:r + 1, :],
                   duo_ref[0, r:r + 1, :], tdo_ref[0, r:r + 1, :],
                   mt, bins, bg, k)
        y2 = jnp.concatenate([ye, yo], axis=0)
        out = lax.dot_general(y2, gd2, (((0,), (0,)), ((), ())),
                              preferred_element_type=jnp.float32)
        o_ref[r * (_BL // 8):(r + 1) * (_BL // 8), :, :] = (
            out.reshape(_BL // 8, 8, 128))


def _eo(x):
    p = x.reshape(-1, 2)
    return (p[:, 0].reshape(_GRID, _RPB, _BL), p[:, 1].reshape(_GRID, _RPB, _BL))


def kernel(weekdays, start_mins, durations, time_diffs, weekday_table,
           hour_table, time_diff_table, duration_table, duration_bins,
           W, b, gamma, beta):
    f32 = jnp.float32
    wde, wdo = _eo(weekdays.astype(jnp.int32))
    sme, smo = _eo(start_mins.astype(jnp.int32))
    due, duo = _eo(durations.astype(f32))
    tde, tdo = _eo(time_diffs.astype(jnp.int32))

    E = jnp.zeros((64, 48), f32)
    E = E.at[0:7, 0:12].set(weekday_table.astype(f32))
    E = E.at[8:32, 12:24].set(hour_table.astype(f32))
    E = E.at[32:42, 26:34].set(duration_table.astype(f32))
    E = E.at[42:50, 34:42].set(time_diff_table.astype(f32))
    E = E.at[50, 24].set(1.0)
    E = E.at[51, 25].set(1.0)
    Wp = jnp.zeros((48, 64), f32).at[0:42, :].set(W.astype(f32))

    McT, bg = pl.pallas_call(
        _fuse_body,
        out_shape=[jax.ShapeDtypeStruct((64, 64), f32),
                   jax.ShapeDtypeStruct((64, 1), f32)],
    )(E, Wp, b.astype(f32).reshape(64, 1),
      beta.astype(f32).reshape(64, 1), gamma.astype(f32).reshape(64, 1))

    g1 = jnp.diag(gamma.astype(f32))
    gd2 = jnp.zeros((128, 128), f32)
    gd2 = gd2.at[0:64, 0:64].set(g1).at[64:128, 64:128].set(g1)
    bins_col = jnp.full((16, 1), jnp.inf, f32).at[0:10, 0].set(
        duration_bins.astype(f32))

    idx_spec = pl.BlockSpec((1, _RPB, _BL), lambda i: (i, 0, 0))
    out3 = pl.pallas_call(
        _main_body,
        grid=(_GRID,),
        in_specs=[idx_spec] * 8 + [
            pl.BlockSpec((64, 64), lambda i: (0, 0)),
            pl.BlockSpec((16, 1), lambda i: (0, 0)),
            pl.BlockSpec((128, 128), lambda i: (0, 0)),
            pl.BlockSpec((64, 1), lambda i: (0, 0)),
        ],
        out_specs=pl.BlockSpec((_OROWS // _GRID, 8, 128), lambda i: (i, 0, 0)),
        out_shape=jax.ShapeDtypeStruct((_OROWS, 8, 128), f32),
    )(wde, wdo, sme, smo, due, duo, tde, tdo, McT, bins_col, gd2, bg)

    return out3.reshape(_B, _L, _H)


# speedup vs baseline: 1.4295x; 1.4295x over previous
import math

import jax
import jax.numpy as jnp
from jax import lax
from jax.experimental import pallas as pl

_B, _L, _H = 4096, 200, 64
_NT = _B * _L
_BL = 2048
_RPB = 4
_GRID = _NT // (2 * _BL * _RPB)
_OROWS = _NT * _H // (8 * 128)


def _fuse_body(e_ref, w_ref, b_ref, be_ref, g_ref, mt_ref, bg_ref):
    mt = lax.dot_general(w_ref[...], e_ref[...], (((0,), (1,)), ((), ())),
                         preferred_element_type=jnp.float32)
    sel = (lax.broadcasted_iota(jnp.int32, (64, 64), 1) == 7).astype(jnp.float32)
    mt = mt + sel * b_ref[...]
    mt_ref[...] = mt - jnp.mean(mt, axis=0, keepdims=True)
    bg_ref[...] = be_ref[...] / g_ref[...]


def _half(wd, sm, du, td, mt, bins, bg, k):
    bf = jnp.bfloat16
    hr = jnp.clip(sm // 60, 0, 23)
    theta = sm.astype(jnp.float32) * jnp.float32(2.0 * math.pi / 1440.0)
    sinv = jnp.sin(theta).astype(bf)
    cosv = jnp.cos(theta).astype(bf)
    ld = jnp.log1p(du)
    cnt = jnp.sum((bins < ld).astype(jnp.int32), axis=0, keepdims=True)
    db = jnp.clip(cnt - 1, 0, 9)
    wd_b = wd.astype(bf)
    hr_b = (hr + 8).astype(bf)
    db_b = (db + 32).astype(bf)
    td_b = (td + 42).astype(bf)
    k0 = k[0:16, :]
    k1 = k[16:32, :]
    k2 = k[32:48, :]
    k3 = k[48:64, :]
    s0 = ((k0 == wd_b) | (k0 == bf(7.0)) | (k0 == hr_b)).astype(bf)
    s1 = (k1 == hr_b).astype(bf)
    s2 = ((k2 == db_b) | (k2 == td_b)).astype(bf)
    s3 = jnp.where(k3 == td_b, bf(1.0),
                   jnp.where(k3 == bf(50.0), sinv,
                             jnp.where(k3 == bf(51.0), cosv, bf(0.0))))
    oh = jnp.concatenate([s0, s1, s2, s3], axis=0)
    ht = lax.dot_general(mt, oh, (((1,), (0,)), ((), ())),
                         preferred_element_type=jnp.float32)
    var = jnp.sum(ht * ht, axis=0, keepdims=True) * jnp.float32(1.0 / 64.0)
    inv = lax.rsqrt(var + 1e-5)
    return ht * inv + bg


def _main_body(wde_ref, wdo_ref, sme_ref, smo_ref, due_ref, duo_ref,
               tde_ref, tdo_ref, mt_ref, bins_ref, gd2_ref, bg_ref, o_ref):
    mt = mt_ref[...].astype(jnp.bfloat16)
    gd2 = gd2_ref[...]
    bg = bg_ref[...]
    bins = bins_ref[...]
    k = lax.broadcasted_iota(jnp.int16, (64, _BL), 0).astype(jnp.bfloat16)
    for r in range(_RPB):
        ye = _half(wde_ref[0, r:r + 1, :], sme_ref[0, r:r + 1, :],
                   due_ref[0, r:r + 1, :], tde_ref[0, r:r + 1, :],
                   mt, bins, bg, k)
        yo = _half(wdo_ref[0, r:r + 1, :], smo_ref[0, r:r + 1, :],
                   duo_ref[0, r:r + 1, :], tdo_ref[0, r:r + 1, :],
                   mt, bins, bg, k)
        y2 = jnp.concatenate([ye, yo], axis=0)
        out = lax.dot_general(y2, gd2, (((0,), (0,)), ((), ())),
                              preferred_element_type=jnp.float32)
        o_ref[r * (_BL // 8):(r + 1) * (_BL // 8), :, :] = (
            out.reshape(_BL // 8, 8, 128))


def _eo(x):
    p = x.reshape(-1, 128)
    return (p[:, 0::2].reshape(_GRID, _RPB, _BL),
            p[:, 1::2].reshape(_GRID, _RPB, _BL))


def kernel(weekdays, start_mins, durations, time_diffs, weekday_table,
           hour_table, time_diff_table, duration_table, duration_bins,
           W, b, gamma, beta):
    f32 = jnp.float32
    wde, wdo = _eo(weekdays.astype(jnp.int32))
    sme, smo = _eo(start_mins.astype(jnp.int32))
    due, duo = _eo(durations.astype(f32))
    tde, tdo = _eo(time_diffs.astype(jnp.int32))

    E = jnp.zeros((64, 48), f32)
    E = E.at[0:7, 0:12].set(weekday_table.astype(f32))
    E = E.at[8:32, 12:24].set(hour_table.astype(f32))
    E = E.at[32:42, 26:34].set(duration_table.astype(f32))
    E = E.at[42:50, 34:42].set(time_diff_table.astype(f32))
    E = E.at[50, 24].set(1.0)
    E = E.at[51, 25].set(1.0)
    Wp = jnp.zeros((48, 64), f32).at[0:42, :].set(W.astype(f32))

    McT, bg = pl.pallas_call(
        _fuse_body,
        out_shape=[jax.ShapeDtypeStruct((64, 64), f32),
                   jax.ShapeDtypeStruct((64, 1), f32)],
    )(E, Wp, b.astype(f32).reshape(64, 1),
      beta.astype(f32).reshape(64, 1), gamma.astype(f32).reshape(64, 1))

    g1 = jnp.diag(gamma.astype(f32))
    gd2 = jnp.zeros((128, 128), f32)
    gd2 = gd2.at[0:64, 0:64].set(g1).at[64:128, 64:128].set(g1)
    bins_col = jnp.full((16, 1), jnp.inf, f32).at[0:10, 0].set(
        duration_bins.astype(f32))

    idx_spec = pl.BlockSpec((1, _RPB, _BL), lambda i: (i, 0, 0))
    out3 = pl.pallas_call(
        _main_body,
        grid=(_GRID,),
        in_specs=[idx_spec] * 8 + [
            pl.BlockSpec((64, 64), lambda i: (0, 0)),
            pl.BlockSpec((16, 1), lambda i: (0, 0)),
            pl.BlockSpec((128, 128), lambda i: (0, 0)),
            pl.BlockSpec((64, 1), lambda i: (0, 0)),
        ],
        out_specs=pl.BlockSpec((_OROWS // _GRID, 8, 128), lambda i: (i, 0, 0)),
        out_shape=jax.ShapeDtypeStruct((_OROWS, 8, 128), f32),
    )(wde, wdo, sme, smo, due, duo, tde, tdo, McT, bins_col, gd2, bg)

    return out3.reshape(_B, _L, _H)


# speedup vs baseline: 3.9500x; 2.7631x over previous
import math

import jax
import jax.numpy as jnp
from jax import lax
from jax.experimental import pallas as pl

_B, _L, _H = 4096, 200, 64
_NT = _B * _L
_BL = 2048
_ROWS = _NT // _BL
_RPB = 8
_GRID = _ROWS // _RPB
_TPB = _RPB * _BL


def _fuse_body(e_ref, w_ref, b_ref, be_ref, g_ref, mt_ref, bg_ref):
    mt = lax.dot_general(w_ref[...], e_ref[...], (((0,), (1,)), ((), ())),
                         preferred_element_type=jnp.float32)
    sel = (lax.broadcasted_iota(jnp.int32, (64, 64), 1) == 7).astype(jnp.float32)
    mt = mt + sel * b_ref[...]
    mt_ref[...] = mt - jnp.mean(mt, axis=0, keepdims=True)
    bg_ref[...] = be_ref[...] / g_ref[...]


def _main_body(wd_ref, sm_ref, du_ref, td_ref, mt_ref, bins_ref, gd_ref,
               bg_ref, o_ref):
    mt = mt_ref[...].astype(jnp.bfloat16)
    gd = gd_ref[...]
    bg = bg_ref[...]
    bins = bins_ref[...]
    k = lax.broadcasted_iota(jnp.int16, (64, _BL), 0).astype(jnp.bfloat16)
    for r in range(_RPB):
        wd = wd_ref[r:r + 1, :]
        sm = sm_ref[r:r + 1, :]
        du = du_ref[r:r + 1, :]
        td = td_ref[r:r + 1, :]
        bf = jnp.bfloat16
        hr = jnp.clip(sm // 60, 0, 23)
        theta = sm.astype(jnp.float32) * jnp.float32(2.0 * math.pi / 1440.0)
        sinv = jnp.sin(theta).astype(bf)
        cosv = jnp.cos(theta).astype(bf)
        ld = jnp.log1p(du)
        cnt = jnp.sum((bins < ld).astype(jnp.int32), axis=0, keepdims=True)
        db = jnp.clip(cnt - 1, 0, 9)
        wd_b = wd.astype(bf)
        hr_b = (hr + 8).astype(bf)
        db_b = (db + 32).astype(bf)
        td_b = (td + 42).astype(bf)
        k0 = k[0:16, :]
        k1 = k[16:32, :]
        k2 = k[32:48, :]
        k3 = k[48:64, :]
        s0 = ((k0 == wd_b) | (k0 == bf(7.0)) | (k0 == hr_b)).astype(bf)
        s1 = (k1 == hr_b).astype(bf)
        s2 = ((k2 == db_b) | (k2 == td_b)).astype(bf)
        s3 = jnp.where(k3 == td_b, bf(1.0),
                       jnp.where(k3 == bf(50.0), sinv,
                                 jnp.where(k3 == bf(51.0), cosv, bf(0.0))))
        oh = jnp.concatenate([s0, s1, s2, s3], axis=0)
        ht = lax.dot_general(mt, oh, (((1,), (0,)), ((), ())),
                             preferred_element_type=jnp.float32)
        var = jnp.sum(ht * ht, axis=0, keepdims=True) * jnp.float32(1.0 / 64.0)
        inv = lax.rsqrt(var + 1e-5)
        y = ht * inv + bg
        out = lax.dot_general(y, gd, (((0,), (0,)), ((), ())),
                              preferred_element_type=jnp.float32)
        o_ref[r * _BL:(r + 1) * _BL, :] = out


def kernel(weekdays, start_mins, durations, time_diffs, weekday_table,
           hour_table, time_diff_table, duration_table, duration_bins,
           W, b, gamma, beta):
    f32 = jnp.float32
    wd2 = weekdays.astype(jnp.int32).reshape(_ROWS, _BL)
    sm2 = start_mins.astype(jnp.int32).reshape(_ROWS, _BL)
    du2 = durations.astype(f32).reshape(_ROWS, _BL)
    td2 = time_diffs.astype(jnp.int32).reshape(_ROWS, _BL)

    E = jnp.zeros((64, 48), f32)
    E = E.at[0:7, 0:12].set(weekday_table.astype(f32))
    E = E.at[8:32, 12:24].set(hour_table.astype(f32))
    E = E.at[32:42, 26:34].set(duration_table.astype(f32))
    E = E.at[42:50, 34:42].set(time_diff_table.astype(f32))
    E = E.at[50, 24].set(1.0)
    E = E.at[51, 25].set(1.0)
    Wp = jnp.zeros((48, 64), f32).at[0:42, :].set(W.astype(f32))

    McT, bg = pl.pallas_call(
        _fuse_body,
        out_shape=[jax.ShapeDtypeStruct((64, 64), f32),
                   jax.ShapeDtypeStruct((64, 1), f32)],
    )(E, Wp, b.astype(f32).reshape(64, 1),
      beta.astype(f32).reshape(64, 1), gamma.astype(f32).reshape(64, 1))

    gd = jnp.diag(gamma.astype(f32))
    bins_col = jnp.full((16, 1), jnp.inf, f32).at[0:10, 0].set(
        duration_bins.astype(f32))

    out2 = pl.pallas_call(
        _main_body,
        grid=(_GRID,),
        in_specs=[
            pl.BlockSpec((_RPB, _BL), lambda i: (i, 0)),
            pl.BlockSpec((_RPB, _BL), lambda i: (i, 0)),
            pl.BlockSpec((_RPB, _BL), lambda i: (i, 0)),
            pl.BlockSpec((_RPB, _BL), lambda i: (i, 0)),
            pl.BlockSpec((64, 64), lambda i: (0, 0)),
            pl.BlockSpec((16, 1), lambda i: (0, 0)),
            pl.BlockSpec((64, 64), lambda i: (0, 0)),
            pl.BlockSpec((64, 1), lambda i: (0, 0)),
        ],
        out_specs=pl.BlockSpec((_TPB, 64), lambda i: (i, 0)),
        out_shape=jax.ShapeDtypeStruct((_NT, 64), f32),
    )(wd2, sm2, du2, td2, McT, bins_col, gd, bg)

    return out2.reshape(_B, _L, _H)
